# Initial kernel scaffold; baseline (speedup 1.0000x reference)
import jax
import jax.numpy as jnp
from jax.experimental import pallas as pl


def kernel(x, edge_index, W1, att_src1, att_dst1, b1, gn_w, gn_b, gn_ms, W2, att_src2, att_dst2, b2):
    raise NotImplementedError("write your pallas kernel here")


# speedup vs baseline: 1.0678x; 1.0678x over previous
import functools
import jax
import jax.numpy as jnp
from jax import lax
from jax.experimental import pallas as pl
from jax.experimental.pallas import tpu as pltpu

_N = 10000
_E = 160000
_DIN = 256
_H = 4
_DOUT = 256
_CHUNK = 1792
_NP = 6 * _CHUNK


def _leaky(v):
    return jnp.where(v > 0, v, 0.2 * v)


def _mm_att_body(x_ref, w_ref, am_ref, b_ref, xw_ref, dots_ref):
    xw_nob = jnp.dot(x_ref[...], w_ref[...], preferred_element_type=jnp.float32)
    dots_ref[...] = jnp.dot(xw_nob, am_ref[...],
                            preferred_element_type=jnp.float32)
    xw_ref[...] = xw_nob + b_ref[...]


def _mm_att(x, w, att_mat, brow, block_k):
    npad, k = x.shape
    kout = w.shape[1]
    grid = (npad // 128,)
    return pl.pallas_call(
        _mm_att_body,
        grid=grid,
        in_specs=[
            pl.BlockSpec((128, k), lambda i: (i, 0)),
            pl.BlockSpec((k, kout), lambda i: (0, 0)),
            pl.BlockSpec((kout, 128), lambda i: (0, 0)),
            pl.BlockSpec((1, kout), lambda i: (0, 0)),
        ],
        out_specs=[
            pl.BlockSpec((128, kout), lambda i: (i, 0)),
            pl.BlockSpec((128, 128), lambda i: (i, 0)),
        ],
        out_shape=[
            jax.ShapeDtypeStruct((npad, kout), jnp.float32),
            jax.ShapeDtypeStruct((npad, 128), jnp.float32),
        ],
    )(x, w, att_mat, brow)


def _stats_body(x_ref, o_ref):
    i = pl.program_id(0)
    rows = lax.broadcasted_iota(jnp.int32, (128, 1), 0) + i * 128
    t = jnp.where(rows < _N, x_ref[...], 0.0)
    s0 = jnp.sum(t, axis=0, keepdims=True)
    s1 = jnp.sum(t * t, axis=0, keepdims=True)

    @pl.when(i == 0)
    def _():
        o_ref[...] = jnp.zeros_like(o_ref)

    o_ref[0:1, :] += s0
    o_ref[1:2, :] += s1


def _stats(x):
    npad, d = x.shape
    return pl.pallas_call(
        _stats_body,
        grid=(npad // 128,),
        in_specs=[pl.BlockSpec((128, d), lambda i: (i, 0))],
        out_specs=pl.BlockSpec((8, d), lambda i: (0, 0)),
        out_shape=jax.ShapeDtypeStruct((8, d), jnp.float32),
    )(x)


def _norm_mm_body(x_ref, sc_ref, sh_ref, w_ref, am_ref, b_ref, xw_ref,
                  dots_ref):
    h = jnp.maximum(x_ref[...] * sc_ref[...] + sh_ref[...], 0.0)
    xw_nob = jnp.dot(h, w_ref[...], preferred_element_type=jnp.float32)
    dots_ref[...] = jnp.dot(xw_nob, am_ref[...],
                            preferred_element_type=jnp.float32)
    xw_ref[...] = xw_nob + b_ref[...]


def _norm_mm(x, scale, shift, w, att_mat, brow):
    npad, k = x.shape
    kout = w.shape[1]
    return pl.pallas_call(
        _norm_mm_body,
        grid=(npad // 128,),
        in_specs=[
            pl.BlockSpec((128, k), lambda i: (i, 0)),
            pl.BlockSpec((1, k), lambda i: (0, 0)),
            pl.BlockSpec((1, k), lambda i: (0, 0)),
            pl.BlockSpec((k, kout), lambda i: (0, 0)),
            pl.BlockSpec((kout, 128), lambda i: (0, 0)),
            pl.BlockSpec((1, kout), lambda i: (0, 0)),
        ],
        out_specs=[
            pl.BlockSpec((128, kout), lambda i: (i, 0)),
            pl.BlockSpec((128, 128), lambda i: (i, 0)),
        ],
        out_shape=[
            jax.ShapeDtypeStruct((npad, kout), jnp.float32),
            jax.ShapeDtypeStruct((npad, 128), jnp.float32),
        ],
    )(x, scale, shift, w, att_mat, brow)


def _edge_phase_xla(xw, a_s, a_d, src, dst, heads, dout):
    n = _N
    loop = jnp.arange(n, dtype=src.dtype)
    src_a = jnp.concatenate([src, loop])
    dst_a = jnp.concatenate([dst, loop])
    alpha = _leaky(a_s[src_a] + a_d[dst_a])
    e = jnp.exp(alpha)
    s = jax.ops.segment_sum(e, dst_a, num_segments=n)
    a = e / (s[dst_a] + 1e-16)
    xw3 = xw[:n].reshape(n, heads, dout)
    out = jax.ops.segment_sum(xw3[src_a] * a[:, :, None], dst_a,
                              num_segments=n)
    return out.reshape(n, heads * dout)


def _build_att_mat(att_src, att_dst, heads, dout):
    k = heads * dout
    m = jnp.zeros((k, 128), jnp.float32)
    rows = jnp.arange(k)
    m = m.at[rows, rows // dout].set(att_src.reshape(-1))
    m = m.at[rows, heads + rows // dout].set(att_dst.reshape(-1))
    return m


def kernel(x, edge_index, W1, att_src1, att_dst1, b1, gn_w, gn_b, gn_ms,
           W2, att_src2, att_dst2, b2):
    src = edge_index[0]
    dst = edge_index[1]

    xpad = jnp.zeros((_NP, _DIN), jnp.float32).at[:_N].set(x)
    am1 = _build_att_mat(att_src1, att_dst1, _H, _DIN)
    xw1, dots1 = _mm_att(xpad, W1, am1, b1.reshape(1, -1), _DIN)
    a_s1 = dots1[:, :_H]
    a_d1 = dots1[:, _H:2 * _H]

    out1 = _edge_phase_xla(xw1, a_s1[:_N], a_d1[:_N], src, dst, _H, _DIN)
    out1 = jnp.zeros((_NP, _H * _DIN), jnp.float32).at[:_N].set(out1)

    st = _stats(out1)
    mean = st[0] / _N
    var = st[1] / _N - mean * mean * (2.0 * gn_ms - gn_ms * gn_ms)
    inv = gn_w / jnp.sqrt(var + 1e-5)
    scale = inv
    shift = gn_b - gn_ms * mean * inv

    am2 = _build_att_mat(att_src2, att_dst2, 1, _DOUT)
    xw2, dots2 = _norm_mm(out1, scale.reshape(1, -1), shift.reshape(1, -1),
                          W2, am2, b2.reshape(1, -1))
    a_s2 = dots2[:, 0:1]
    a_d2 = dots2[:, 1:2]

    out2 = _edge_phase_xla(xw2, a_s2[:_N], a_d2[:_N], src, dst, 1, _DOUT)
    return out2


# speedup vs baseline: 3.9300x; 3.6804x over previous
import functools
import jax
import jax.numpy as jnp
from jax import lax
from jax.experimental import pallas as pl
from jax.experimental.pallas import tpu as pltpu
from jax.experimental.pallas import tpu_sc as plsc

_N = 10000
_E = 160000
_DIN = 256
_H = 4
_DOUT = 256
_CHUNK = 256
_CHUNK2 = 1024
_NP = 10240


def _leaky(v):
    return jnp.where(v > 0, v, 0.2 * v)


def _mm_att_body(x_ref, w_ref, am_ref, b_ref, xw_ref, dots_ref):
    xw_nob = jnp.dot(x_ref[...], w_ref[...], preferred_element_type=jnp.float32)
    dots_ref[...] = jnp.dot(xw_nob, am_ref[...],
                            preferred_element_type=jnp.float32)
    xw_ref[...] = xw_nob + b_ref[...]


def _mm_att(x, w, att_mat, brow, block_k):
    npad, k = x.shape
    kout = w.shape[1]
    grid = (npad // 128,)
    return pl.pallas_call(
        _mm_att_body,
        grid=grid,
        in_specs=[
            pl.BlockSpec((128, k), lambda i: (i, 0)),
            pl.BlockSpec((k, kout), lambda i: (0, 0)),
            pl.BlockSpec((kout, 128), lambda i: (0, 0)),
            pl.BlockSpec((1, kout), lambda i: (0, 0)),
        ],
        out_specs=[
            pl.BlockSpec((128, kout), lambda i: (i, 0)),
            pl.BlockSpec((128, 128), lambda i: (i, 0)),
        ],
        out_shape=[
            jax.ShapeDtypeStruct((npad, kout), jnp.float32),
            jax.ShapeDtypeStruct((npad, 128), jnp.float32),
        ],
    )(x, w, att_mat, brow)


def _stats_body(x_ref, o_ref):
    i = pl.program_id(0)
    rows = lax.broadcasted_iota(jnp.int32, (128, 1), 0) + i * 128
    t = jnp.where(rows < _N, x_ref[...], 0.0)
    s0 = jnp.sum(t, axis=0, keepdims=True)
    s1 = jnp.sum(t * t, axis=0, keepdims=True)

    @pl.when(i == 0)
    def _():
        o_ref[...] = jnp.zeros_like(o_ref)

    o_ref[0:1, :] += s0
    o_ref[1:2, :] += s1


def _stats(x):
    npad, d = x.shape
    return pl.pallas_call(
        _stats_body,
        grid=(npad // 128,),
        in_specs=[pl.BlockSpec((128, d), lambda i: (i, 0))],
        out_specs=pl.BlockSpec((8, d), lambda i: (0, 0)),
        out_shape=jax.ShapeDtypeStruct((8, d), jnp.float32),
    )(x)


def _norm_mm_body(x_ref, sc_ref, sh_ref, w_ref, am_ref, b_ref, xw_ref,
                  dots_ref):
    h = jnp.maximum(x_ref[...] * sc_ref[...] + sh_ref[...], 0.0)
    xw_nob = jnp.dot(h, w_ref[...], preferred_element_type=jnp.float32)
    dots_ref[...] = jnp.dot(xw_nob, am_ref[...],
                            preferred_element_type=jnp.float32)
    xw_ref[...] = xw_nob + b_ref[...]


def _norm_mm(x, scale, shift, w, att_mat, brow):
    npad, k = x.shape
    kout = w.shape[1]
    return pl.pallas_call(
        _norm_mm_body,
        grid=(npad // 128,),
        in_specs=[
            pl.BlockSpec((128, k), lambda i: (i, 0)),
            pl.BlockSpec((1, k), lambda i: (0, 0)),
            pl.BlockSpec((1, k), lambda i: (0, 0)),
            pl.BlockSpec((k, kout), lambda i: (0, 0)),
            pl.BlockSpec((kout, 128), lambda i: (0, 0)),
            pl.BlockSpec((1, kout), lambda i: (0, 0)),
        ],
        out_specs=[
            pl.BlockSpec((128, kout), lambda i: (i, 0)),
            pl.BlockSpec((128, 128), lambda i: (i, 0)),
        ],
        out_shape=[
            jax.ShapeDtypeStruct((npad, kout), jnp.float32),
            jax.ShapeDtypeStruct((npad, 128), jnp.float32),
        ],
    )(x, scale, shift, w, att_mat, brow)


def _leaky_exp(v):
    return jnp.exp(jnp.where(v > 0, v, 0.2 * v))


def _make_edge_sc(heads, dout, chunk):
    hd = heads * dout
    ept = _E // 16
    ng = ept // 16
    own = chunk // 16
    sh = chunk * heads
    n_pass = _NP // (2 * chunk)
    pub = ept + 16
    blk = 1024
    mesh = plsc.VectorSubcoreMesh(core_axis_name="c", subcore_axis_name="s")

    @functools.partial(
        pl.kernel, mesh=mesh,
        out_type=[jax.ShapeDtypeStruct((_NP, hd), jnp.float32),
                  jax.ShapeDtypeStruct((32 * sh,), jnp.float32),
                  jax.ShapeDtypeStruct((32 * pub,), jnp.int32),
                  jax.ShapeDtypeStruct((32 * pub,), jnp.int32),
                  jax.ShapeDtypeStruct((32 * 16,), jnp.int32)],
        compiler_params=pltpu.CompilerParams(needs_layout_passes=False),
        scratch_types=[
            pltpu.VMEM((ept,), jnp.int32),
            pltpu.VMEM((ept,), jnp.int32),
            pltpu.VMEM((_NP * heads,), jnp.float32),
            pltpu.VMEM((sh,), jnp.float32),
            pltpu.VMEM((sh,), jnp.float32),
            pltpu.VMEM((pub,), jnp.int32),
            pltpu.VMEM((pub,), jnp.int32),
            pltpu.VMEM((16, hd), jnp.float32),
            pltpu.VMEM((own, hd), jnp.float32),
            pltpu.VMEM((own * heads,), jnp.float32),
            pltpu.VMEM((16 * heads,), jnp.float32),
            pltpu.VMEM((2048,), jnp.float32),
            pltpu.VMEM((128,), jnp.float32),
            pltpu.VMEM((16,), jnp.int32),
            pltpu.VMEM((256,), jnp.int32),
            pltpu.VMEM((blk,), jnp.int32),
            pltpu.VMEM((blk,), jnp.int32),
            pltpu.VMEM((blk + 16,), jnp.int32),
            pltpu.VMEM((blk + 32,), jnp.int32),
            pltpu.VMEM_SHARED((sh,), jnp.float32),
            pltpu.SemaphoreType.DMA,
        ])
    def k(src_h, dst_h, as_h, ad_h, xw_h,
          out_h, parts_h, pubs_h, pubd_h, cnts_h,
          src_t, dst_t, as_t, ad_t, rs_t, srcc, dstc, rows, out_own,
          aws, awb, mbuf, vbuf, cbuf, cntbuf, psrc, pdst, osrc, orel,
          s_sh, sem):
        cid = lax.axis_index("c")
        sid = lax.axis_index("s")
        ebase = sid * ept
        pltpu.sync_copy(src_h.at[pl.ds(ebase, ept)], src_t)
        pltpu.sync_copy(dst_h.at[pl.ds(ebase, ept)], dst_t)
        pltpu.sync_copy(as_h, as_t)

        zi = jnp.zeros((16,), jnp.int32)
        zf = jnp.zeros((16,), jnp.float32)

        def zb(i, c):
            srcc[pl.ds(i * 16, 16)] = zi
            dstc[pl.ds(i * 16, 16)] = zi
            return c
        lax.fori_loop(0, pub // 16, zb, 0)

        def zo(i, c):
            osrc[pl.ds(i * 16, 16)] = zi
            orel[pl.ds(i * 16, 16)] = zi
            return c
        lax.fori_loop(0, (blk + 16) // 16, zo, 0)

        def pass_body(p, carry):
            lo = (2 * p + cid) * chunk
            pltpu.sync_copy(ad_h.at[pl.ds(lo * heads, sh)], ad_t)

            def zs(i, c):
                rs_t[pl.ds(i * 16, 16)] = zf
                return c
            lax.fori_loop(0, sh // 16, zs, 0)

            def ea(i, cnt):
                sl = pl.ds(i * 16, 16)
                sv = src_t[sl]
                rel = dst_t[sl] - lo
                m = (rel >= 0) & (rel < chunk)
                pc = plsc.all_reduce_population_count(m)

                @pl.when(pc[0] > 0)
                def _():
                    relc = jnp.where(m, rel, 0)
                    svc = jnp.where(m, sv, 0)
                    for h in range(heads):
                        asv = plsc.load_gather(as_t, [svc * heads + h])
                        adv = plsc.load_gather(ad_t, [relc * heads + h])
                        e = _leaky_exp(asv + adv)
                        plsc.addupdate_scatter(rs_t, [relc * heads + h], e,
                                               mask=m)
                    plsc.store_compressed(srcc.at[pl.ds(cnt, 16)], svc,
                                          mask=m)
                    plsc.store_compressed(dstc.at[pl.ds(cnt, 16)], relc,
                                          mask=m)
                return cnt + pc[0]
            cnt = lax.fori_loop(0, ng, ea, jnp.int32(0))

            def slp(i, c):
                off = sid * (own * heads) + i * 16
                e = _leaky_exp(as_t[pl.ds(lo * heads + off, 16)]
                               + ad_t[pl.ds(off, 16)])
                rs_t[pl.ds(off, 16)] = rs_t[pl.ds(off, 16)] + e
                return c
            lax.fori_loop(0, (own * heads) // 16, slp, 0)

            n_sl = sh // 128
            wid = cid * 16 + sid
            pltpu.sync_copy(rs_t, parts_h.at[pl.ds(wid * sh, sh)])
            plsc.subcore_barrier()
            for q in range((n_sl + 15) // 16):
                s_idx = sid + q * 16

                @pl.when(s_idx < n_sl)
                def _():
                    cps = [
                        pltpu.async_copy(
                            parts_h.at[pl.ds((cid * 16 + t) * sh
                                             + s_idx * 128, 128)],
                            mbuf.at[pl.ds(t * 128, 128)], sem)
                        for t in range(16)
                    ]
                    for cp in cps:
                        cp.wait()

                    def red(i, c):
                        sl = pl.ds(i * 16, 16)
                        acc = mbuf[pl.ds(i * 16, 16)]
                        for t in range(1, 16):
                            acc = acc + mbuf[pl.ds(t * 128 + i * 16, 16)]
                        vbuf[sl] = 1.0 / (acc + 1e-16)
                        return c
                    lax.fori_loop(0, 8, red, 0)
                    pltpu.sync_copy(vbuf, s_sh.at[pl.ds(s_idx * 128, 128)])
            plsc.subcore_barrier()
            pltpu.sync_copy(s_sh, rs_t)

            def swb(i, c):
                off = sid * (own * heads) + i * 16
                e = _leaky_exp(as_t[pl.ds(lo * heads + off, 16)]
                               + ad_t[pl.ds(off, 16)])
                aws[pl.ds(i * 16, 16)] = e * rs_t[pl.ds(off, 16)]
                return c
            lax.fori_loop(0, (own * heads) // 16, swb, 0)

            wid2 = cid * 16 + sid
            pbase = wid2 * pub

            def pb(b, c):
                pltpu.sync_copy(srcc.at[pl.ds(b * blk, blk)],
                                pubs_h.at[pl.ds(pbase + b * blk, blk)])
                pltpu.sync_copy(dstc.at[pl.ds(b * blk, blk)],
                                pubd_h.at[pl.ds(pbase + b * blk, blk)])
                return c
            lax.fori_loop(0, (cnt + blk - 1) // blk, pb, 0)
            cbuf[...] = jnp.broadcast_to(cnt, (16,))
            pltpu.sync_copy(cbuf, cnts_h.at[pl.ds(wid2 * 16, 16)])
            plsc.subcore_barrier()

            r0 = lo + sid * own

            def ib(g, c):
                pltpu.sync_copy(xw_h.at[pl.ds(r0 + g * 16, 16)], rows)

                def scale_row(rr, c2):
                    for h in range(heads):
                        wi = jnp.broadcast_to((g * 16 + rr) * heads + h,
                                              (16,))
                        wv = plsc.load_gather(aws, [wi])
                        for cc in range(dout // 16):
                            sl2 = pl.ds(h * dout + cc * 16, 16)
                            out_own[g * 16 + rr, sl2] = rows[rr, sl2] * wv
                    return c2
                lax.fori_loop(0, 16, scale_row, 0)
                return c
            lax.fori_loop(0, own // 16, ib, 0)

            olo = sid * own
            pltpu.sync_copy(cnts_h.at[pl.ds(cid * 256, 256)], cntbuf)

            def per_src_tile(t, c):
                ctv = cntbuf[pl.ds(t * 16, 16)]
                cntt = ctv[0]
                base = (cid * 16 + t) * pub

                def per_blk(b, c2):
                    pltpu.sync_copy(pubs_h.at[pl.ds(base + b * blk, blk)],
                                    psrc)
                    pltpu.sync_copy(pubd_h.at[pl.ds(base + b * blk, blk)],
                                    pdst)
                    eib = jnp.minimum(cntt - b * blk, blk)

                    def fl(g, oc):
                        sl = pl.ds(g * 16, 16)
                        relv = pdst[sl]
                        srcv = psrc[sl]
                        lane = lax.iota(jnp.int32, 16)
                        m3 = ((g * 16 + lane) < eib) & (relv >= olo) \
                            & (relv < olo + own)
                        pc3 = plsc.all_reduce_population_count(m3)

                        @pl.when(pc3[0] > 0)
                        def _():
                            plsc.store_compressed(
                                osrc.at[pl.ds(oc, 16)],
                                jnp.where(m3, srcv, 0), mask=m3)
                            plsc.store_compressed(
                                orel.at[pl.ds(oc, 16)],
                                jnp.where(m3, relv - olo, 0), mask=m3)
                        return oc + pc3[0]
                    ocnt = lax.fori_loop(0, (eib + 15) // 16, fl,
                                         jnp.int32(0))

                    def eb(g, c3):
                        off = g * 16
                        sl = pl.ds(off, 16)
                        sv = osrc[sl]
                        rel = orel[sl]
                        lane = lax.iota(jnp.int32, 16)
                        m2 = (off + lane) < ocnt
                        for h in range(heads):
                            asv = plsc.load_gather(as_t, [sv * heads + h])
                            adv = plsc.load_gather(
                                ad_t, [(rel + olo) * heads + h])
                            rsv = plsc.load_gather(
                                rs_t, [(rel + olo) * heads + h])
                            e = _leaky_exp(asv + adv)
                            awb[pl.ds(h * 16, 16)] = jnp.where(
                                m2, e * rsv, 0.0)
                        pltpu.async_copy(xw_h.at[sv], rows, sem).wait()

                        def acc_row(rr, c4):
                            rv = orel[pl.ds(off + rr, 16)]
                            rloc = rv[0]
                            for h in range(heads):
                                wi = jnp.broadcast_to(h * 16 + rr, (16,))
                                wv = plsc.load_gather(awb, [wi])
                                for cc in range(dout // 16):
                                    sl2 = pl.ds(h * dout + cc * 16, 16)
                                    out_own[rloc, sl2] = (
                                        out_own[rloc, sl2]
                                        + rows[rr, sl2] * wv)
                            return c4
                        lax.fori_loop(0, 16, acc_row, 0)
                        return c3
                    lax.fori_loop(0, (ocnt + 15) // 16, eb, 0)
                    return c2
                lax.fori_loop(0, (cntt + blk - 1) // blk, per_blk, 0)
                return c
            lax.fori_loop(0, 16, per_src_tile, 0)

            pltpu.sync_copy(out_own, out_h.at[pl.ds(r0, own)])
            plsc.subcore_barrier()
            return carry
        lax.fori_loop(0, n_pass, pass_body, 0)

    return k


_edge_sc1 = _make_edge_sc(_H, _DIN, _CHUNK)
_edge_sc2 = _make_edge_sc(1, _DOUT, _CHUNK2)


def _edge_phase_xla(xw, a_s, a_d, src, dst, heads, dout):
    n = _N
    loop = jnp.arange(n, dtype=src.dtype)
    src_a = jnp.concatenate([src, loop])
    dst_a = jnp.concatenate([dst, loop])
    alpha = _leaky(a_s[src_a] + a_d[dst_a])
    e = jnp.exp(alpha)
    s = jax.ops.segment_sum(e, dst_a, num_segments=n)
    a = e / (s[dst_a] + 1e-16)
    xw3 = xw[:n].reshape(n, heads, dout)
    out = jax.ops.segment_sum(xw3[src_a] * a[:, :, None], dst_a,
                              num_segments=n)
    return out.reshape(n, heads * dout)


def _build_att_mat(att_src, att_dst, heads, dout):
    k = heads * dout
    m = jnp.zeros((k, 128), jnp.float32)
    rows = jnp.arange(k)
    m = m.at[rows, rows // dout].set(att_src.reshape(-1))
    m = m.at[rows, heads + rows // dout].set(att_dst.reshape(-1))
    return m


def kernel(x, edge_index, W1, att_src1, att_dst1, b1, gn_w, gn_b, gn_ms,
           W2, att_src2, att_dst2, b2):
    src = edge_index[0]
    dst = edge_index[1]

    xpad = jnp.zeros((_NP, _DIN), jnp.float32).at[:_N].set(x)
    am1 = _build_att_mat(att_src1, att_dst1, _H, _DIN)
    xw1, dots1 = _mm_att(xpad, W1, am1, b1.reshape(1, -1), _DIN)
    a_s1 = dots1[:, :_H].reshape(-1)
    a_d1 = dots1[:, _H:2 * _H].reshape(-1)

    out1 = _edge_sc1(src, dst, a_s1, a_d1, xw1)[0]

    st = _stats(out1)
    mean = st[0] / _N
    var = st[1] / _N - mean * mean * (2.0 * gn_ms - gn_ms * gn_ms)
    inv = gn_w / jnp.sqrt(var + 1e-5)
    scale = inv
    shift = gn_b - gn_ms * mean * inv

    am2 = _build_att_mat(att_src2, att_dst2, 1, _DOUT)
    xw2, dots2 = _norm_mm(out1, scale.reshape(1, -1), shift.reshape(1, -1),
                          W2, am2, b2.reshape(1, -1))
    a_s2 = dots2[:, 0]
    a_d2 = dots2[:, 1]

    out2 = _edge_sc2(src, dst, a_s2, a_d2, xw2)[0]
    return out2[:_N]


# speedup vs baseline: 4.0716x; 1.0360x over previous
import functools
import jax
import jax.numpy as jnp
from jax import lax
from jax.experimental import pallas as pl
from jax.experimental.pallas import tpu as pltpu
from jax.experimental.pallas import tpu_sc as plsc

_N = 10000
_E = 160000
_DIN = 256
_H = 4
_DOUT = 256
_CHUNK = 256
_CHUNK2 = 1024
_NP = 10240


def _leaky(v):
    return jnp.where(v > 0, v, 0.2 * v)


def _mm_att_body(x_ref, w_ref, am_ref, b_ref, xw_ref, dots_ref):
    xw_nob = jnp.dot(x_ref[...], w_ref[...], preferred_element_type=jnp.float32)
    dots_ref[...] = jnp.dot(xw_nob, am_ref[...],
                            preferred_element_type=jnp.float32)
    xw_ref[...] = xw_nob + b_ref[...]


def _mm_att(x, w, att_mat, brow, block_k):
    npad, k = x.shape
    kout = w.shape[1]
    grid = (npad // 128,)
    return pl.pallas_call(
        _mm_att_body,
        grid=grid,
        in_specs=[
            pl.BlockSpec((128, k), lambda i: (i, 0)),
            pl.BlockSpec((k, kout), lambda i: (0, 0)),
            pl.BlockSpec((kout, 128), lambda i: (0, 0)),
            pl.BlockSpec((1, kout), lambda i: (0, 0)),
        ],
        out_specs=[
            pl.BlockSpec((128, kout), lambda i: (i, 0)),
            pl.BlockSpec((128, 128), lambda i: (i, 0)),
        ],
        out_shape=[
            jax.ShapeDtypeStruct((npad, kout), jnp.float32),
            jax.ShapeDtypeStruct((npad, 128), jnp.float32),
        ],
    )(x, w, att_mat, brow)


def _stats_body(x_ref, o_ref):
    i = pl.program_id(0)
    rows = lax.broadcasted_iota(jnp.int32, (128, 1), 0) + i * 128
    t = jnp.where(rows < _N, x_ref[...], 0.0)
    s0 = jnp.sum(t, axis=0, keepdims=True)
    s1 = jnp.sum(t * t, axis=0, keepdims=True)

    @pl.when(i == 0)
    def _():
        o_ref[...] = jnp.zeros_like(o_ref)

    o_ref[0:1, :] += s0
    o_ref[1:2, :] += s1


def _stats(x):
    npad, d = x.shape
    return pl.pallas_call(
        _stats_body,
        grid=(npad // 128,),
        in_specs=[pl.BlockSpec((128, d), lambda i: (i, 0))],
        out_specs=pl.BlockSpec((8, d), lambda i: (0, 0)),
        out_shape=jax.ShapeDtypeStruct((8, d), jnp.float32),
    )(x)


def _norm_mm_body(x_ref, sc_ref, sh_ref, w_ref, am_ref, b_ref, xw_ref,
                  dots_ref):
    h = jnp.maximum(x_ref[...] * sc_ref[...] + sh_ref[...], 0.0)
    xw_nob = jnp.dot(h, w_ref[...], preferred_element_type=jnp.float32)
    dots_ref[...] = jnp.dot(xw_nob, am_ref[...],
                            preferred_element_type=jnp.float32)
    xw_ref[...] = xw_nob + b_ref[...]


def _norm_mm(x, scale, shift, w, att_mat, brow):
    npad, k = x.shape
    kout = w.shape[1]
    return pl.pallas_call(
        _norm_mm_body,
        grid=(npad // 128,),
        in_specs=[
            pl.BlockSpec((128, k), lambda i: (i, 0)),
            pl.BlockSpec((1, k), lambda i: (0, 0)),
            pl.BlockSpec((1, k), lambda i: (0, 0)),
            pl.BlockSpec((k, kout), lambda i: (0, 0)),
            pl.BlockSpec((kout, 128), lambda i: (0, 0)),
            pl.BlockSpec((1, kout), lambda i: (0, 0)),
        ],
        out_specs=[
            pl.BlockSpec((128, kout), lambda i: (i, 0)),
            pl.BlockSpec((128, 128), lambda i: (i, 0)),
        ],
        out_shape=[
            jax.ShapeDtypeStruct((npad, kout), jnp.float32),
            jax.ShapeDtypeStruct((npad, 128), jnp.float32),
        ],
    )(x, scale, shift, w, att_mat, brow)


def _leaky_exp(v):
    return jnp.exp(jnp.where(v > 0, v, 0.2 * v))


def _make_edge_sc(heads, dout, chunk):
    hd = heads * dout
    ept = _E // 16
    ng = ept // 16
    own = chunk // 16
    sh = chunk * heads
    n_pass = _NP // (2 * chunk)
    pub = ept + 16
    blk = 1024
    mesh = plsc.VectorSubcoreMesh(core_axis_name="c", subcore_axis_name="s")

    @functools.partial(
        pl.kernel, mesh=mesh,
        out_type=[jax.ShapeDtypeStruct((_NP, hd), jnp.float32),
                  jax.ShapeDtypeStruct((32 * sh,), jnp.float32),
                  jax.ShapeDtypeStruct((32 * pub,), jnp.int32),
                  jax.ShapeDtypeStruct((32 * pub,), jnp.int32),
                  jax.ShapeDtypeStruct((32 * 16,), jnp.int32)],
        compiler_params=pltpu.CompilerParams(needs_layout_passes=False),
        scratch_types=[
            pltpu.VMEM((ept,), jnp.int32),
            pltpu.VMEM((ept,), jnp.int32),
            pltpu.VMEM((_NP * heads,), jnp.float32),
            pltpu.VMEM((sh,), jnp.float32),
            pltpu.VMEM((sh,), jnp.float32),
            pltpu.VMEM((pub,), jnp.int32),
            pltpu.VMEM((pub,), jnp.int32),
            pltpu.VMEM((16, hd), jnp.float32),
            pltpu.VMEM((own, hd), jnp.float32),
            pltpu.VMEM((own * heads,), jnp.float32),
            pltpu.VMEM((16 * heads,), jnp.float32),
            pltpu.VMEM((2048,), jnp.float32),
            pltpu.VMEM((128,), jnp.float32),
            pltpu.VMEM((16,), jnp.int32),
            pltpu.VMEM((256,), jnp.int32),
            pltpu.VMEM((blk,), jnp.int32),
            pltpu.VMEM((blk,), jnp.int32),
            pltpu.VMEM((blk + 16,), jnp.int32),
            pltpu.VMEM((blk + 32,), jnp.int32),
            pltpu.VMEM_SHARED((sh,), jnp.float32),
            pltpu.SemaphoreType.DMA,
        ])
    def k(src_h, dst_h, as_h, ad_h, xw_h,
          out_h, parts_h, pubs_h, pubd_h, cnts_h,
          src_t, dst_t, as_t, ad_t, rs_t, srcc, dstc, rows, out_own,
          aws, awb, mbuf, vbuf, cbuf, cntbuf, psrc, pdst, osrc, orel,
          s_sh, sem):
        cid = lax.axis_index("c")
        sid = lax.axis_index("s")
        ebase = sid * ept
        pltpu.sync_copy(src_h.at[pl.ds(ebase, ept)], src_t)
        pltpu.sync_copy(dst_h.at[pl.ds(ebase, ept)], dst_t)
        pltpu.sync_copy(as_h, as_t)

        zi = jnp.zeros((16,), jnp.int32)
        zf = jnp.zeros((16,), jnp.float32)

        def zb(i, c):
            srcc[pl.ds(i * 16, 16)] = zi
            dstc[pl.ds(i * 16, 16)] = zi
            return c
        lax.fori_loop(0, pub // 16, zb, 0)

        def zo(i, c):
            osrc[pl.ds(i * 16, 16)] = zi
            orel[pl.ds(i * 16, 16)] = zi
            return c
        lax.fori_loop(0, (blk + 16) // 16, zo, 0)

        def pass_body(p, carry):
            lo = (2 * p + cid) * chunk
            pltpu.sync_copy(ad_h.at[pl.ds(lo * heads, sh)], ad_t)

            def zs(i, c):
                rs_t[pl.ds(i * 16, 16)] = zf
                return c
            lax.fori_loop(0, sh // 16, zs, 0)

            def ea(i, cnt):
                sl = pl.ds(i * 16, 16)
                sv = src_t[sl]
                rel = dst_t[sl] - lo
                m = (rel >= 0) & (rel < chunk)
                pc = plsc.all_reduce_population_count(m)

                @pl.when(pc[0] > 0)
                def _():
                    relc = jnp.where(m, rel, 0)
                    svc = jnp.where(m, sv, 0)
                    for h in range(heads):
                        asv = plsc.load_gather(as_t, [svc * heads + h])
                        adv = plsc.load_gather(ad_t, [relc * heads + h])
                        e = _leaky_exp(asv + adv)
                        plsc.addupdate_scatter(rs_t, [relc * heads + h], e,
                                               mask=m)
                    plsc.store_compressed(srcc.at[pl.ds(cnt, 16)], svc,
                                          mask=m)
                    plsc.store_compressed(dstc.at[pl.ds(cnt, 16)], relc,
                                          mask=m)
                return cnt + pc[0]
            cnt = lax.fori_loop(0, ng, ea, jnp.int32(0))

            def slp(i, c):
                off = sid * (own * heads) + i * 16
                e = _leaky_exp(as_t[pl.ds(lo * heads + off, 16)]
                               + ad_t[pl.ds(off, 16)])
                rs_t[pl.ds(off, 16)] = rs_t[pl.ds(off, 16)] + e
                return c
            lax.fori_loop(0, (own * heads) // 16, slp, 0)

            n_sl = sh // 128
            wid = cid * 16 + sid
            pltpu.sync_copy(rs_t, parts_h.at[pl.ds(wid * sh, sh)])
            plsc.subcore_barrier()
            for q in range((n_sl + 15) // 16):
                s_idx = sid + q * 16

                @pl.when(s_idx < n_sl)
                def _():
                    cps = [
                        pltpu.async_copy(
                            parts_h.at[pl.ds((cid * 16 + t) * sh
                                             + s_idx * 128, 128)],
                            mbuf.at[pl.ds(t * 128, 128)], sem)
                        for t in range(16)
                    ]
                    for cp in cps:
                        cp.wait()

                    def red(i, c):
                        sl = pl.ds(i * 16, 16)
                        acc = mbuf[pl.ds(i * 16, 16)]
                        for t in range(1, 16):
                            acc = acc + mbuf[pl.ds(t * 128 + i * 16, 16)]
                        vbuf[sl] = 1.0 / (acc + 1e-16)
                        return c
                    lax.fori_loop(0, 8, red, 0)
                    pltpu.sync_copy(vbuf, s_sh.at[pl.ds(s_idx * 128, 128)])
            plsc.subcore_barrier()
            pltpu.sync_copy(s_sh, rs_t)

            def swb(i, c):
                off = sid * (own * heads) + i * 16
                e = _leaky_exp(as_t[pl.ds(lo * heads + off, 16)]
                               + ad_t[pl.ds(off, 16)])
                aws[pl.ds(i * 16, 16)] = e * rs_t[pl.ds(off, 16)]
                return c
            lax.fori_loop(0, (own * heads) // 16, swb, 0)

            wid2 = cid * 16 + sid
            pbase = wid2 * pub

            def pb(b, c):
                pltpu.sync_copy(srcc.at[pl.ds(b * blk, blk)],
                                pubs_h.at[pl.ds(pbase + b * blk, blk)])
                pltpu.sync_copy(dstc.at[pl.ds(b * blk, blk)],
                                pubd_h.at[pl.ds(pbase + b * blk, blk)])
                return c
            lax.fori_loop(0, (cnt + blk - 1) // blk, pb, 0)
            cbuf[...] = jnp.broadcast_to(cnt, (16,))
            pltpu.sync_copy(cbuf, cnts_h.at[pl.ds(wid2 * 16, 16)])
            plsc.subcore_barrier()

            r0 = lo + sid * own

            def ib(g, c):
                pltpu.sync_copy(xw_h.at[pl.ds(r0 + g * 16, 16)], rows)

                def scale_row(rr, c2):
                    for h in range(heads):
                        wi = jnp.broadcast_to((g * 16 + rr) * heads + h,
                                              (16,))
                        wv = plsc.load_gather(aws, [wi])
                        for cc in range(dout // 16):
                            sl2 = pl.ds(h * dout + cc * 16, 16)
                            out_own[g * 16 + rr, sl2] = rows[rr, sl2] * wv
                    return c2
                lax.fori_loop(0, 16, scale_row, 0)
                return c
            lax.fori_loop(0, own // 16, ib, 0)

            olo = sid * own
            pltpu.sync_copy(cnts_h.at[pl.ds(cid * 256, 256)], cntbuf)

            def per_src_tile(t, c):
                ctv = cntbuf[pl.ds(t * 16, 16)]
                cntt = ctv[0]
                base = (cid * 16 + t) * pub

                def per_blk(b, c2):
                    cp1 = pltpu.async_copy(
                        pubs_h.at[pl.ds(base + b * blk, blk)], psrc, sem)
                    cp2 = pltpu.async_copy(
                        pubd_h.at[pl.ds(base + b * blk, blk)], pdst, sem)
                    cp1.wait()
                    cp2.wait()
                    eib = jnp.minimum(cntt - b * blk, blk)

                    def fl(g, oc):
                        sl = pl.ds(g * 16, 16)
                        relv = pdst[sl]
                        srcv = psrc[sl]
                        lane = lax.iota(jnp.int32, 16)
                        m3 = ((g * 16 + lane) < eib) & (relv >= olo) \
                            & (relv < olo + own)
                        pc3 = plsc.all_reduce_population_count(m3)

                        @pl.when(pc3[0] > 0)
                        def _():
                            plsc.store_compressed(
                                osrc.at[pl.ds(oc, 16)],
                                jnp.where(m3, srcv, 0), mask=m3)
                            plsc.store_compressed(
                                orel.at[pl.ds(oc, 16)],
                                jnp.where(m3, relv - olo, 0), mask=m3)
                        return oc + pc3[0]
                    ocnt = lax.fori_loop(0, (eib + 15) // 16, fl,
                                         jnp.int32(0))

                    def eb(g, c3):
                        off = g * 16
                        sl = pl.ds(off, 16)
                        sv = osrc[sl]
                        rel = orel[sl]
                        gcp = pltpu.async_copy(xw_h.at[sv], rows, sem)
                        lane = lax.iota(jnp.int32, 16)
                        m2 = (off + lane) < ocnt
                        for h in range(heads):
                            asv = plsc.load_gather(as_t, [sv * heads + h])
                            adv = plsc.load_gather(
                                ad_t, [(rel + olo) * heads + h])
                            rsv = plsc.load_gather(
                                rs_t, [(rel + olo) * heads + h])
                            e = _leaky_exp(asv + adv)
                            awb[pl.ds(h * 16, 16)] = jnp.where(
                                m2, e * rsv, 0.0)
                        gcp.wait()

                        def acc_row(rr, c4):
                            rv = orel[pl.ds(off + rr, 16)]
                            rloc = rv[0]
                            for h in range(heads):
                                wi = jnp.broadcast_to(h * 16 + rr, (16,))
                                wv = plsc.load_gather(awb, [wi])
                                for cc in range(dout // 16):
                                    sl2 = pl.ds(h * dout + cc * 16, 16)
                                    out_own[rloc, sl2] = (
                                        out_own[rloc, sl2]
                                        + rows[rr, sl2] * wv)
                            return c4
                        lax.fori_loop(0, 16, acc_row, 0)
                        return c3
                    lax.fori_loop(0, (ocnt + 15) // 16, eb, 0)
                    return c2
                lax.fori_loop(0, (cntt + blk - 1) // blk, per_blk, 0)
                return c
            lax.fori_loop(0, 16, per_src_tile, 0)

            pltpu.sync_copy(out_own, out_h.at[pl.ds(r0, own)])
            plsc.subcore_barrier()
            return carry
        lax.fori_loop(0, n_pass, pass_body, 0)

    return k


_edge_sc1 = _make_edge_sc(_H, _DIN, _CHUNK)
_edge_sc2 = _make_edge_sc(1, _DOUT, _CHUNK2)


def _edge_phase_xla(xw, a_s, a_d, src, dst, heads, dout):
    n = _N
    loop = jnp.arange(n, dtype=src.dtype)
    src_a = jnp.concatenate([src, loop])
    dst_a = jnp.concatenate([dst, loop])
    alpha = _leaky(a_s[src_a] + a_d[dst_a])
    e = jnp.exp(alpha)
    s = jax.ops.segment_sum(e, dst_a, num_segments=n)
    a = e / (s[dst_a] + 1e-16)
    xw3 = xw[:n].reshape(n, heads, dout)
    out = jax.ops.segment_sum(xw3[src_a] * a[:, :, None], dst_a,
                              num_segments=n)
    return out.reshape(n, heads * dout)


def _build_att_mat(att_src, att_dst, heads, dout):
    k = heads * dout
    m = jnp.zeros((k, 128), jnp.float32)
    rows = jnp.arange(k)
    m = m.at[rows, rows // dout].set(att_src.reshape(-1))
    m = m.at[rows, heads + rows // dout].set(att_dst.reshape(-1))
    return m


def kernel(x, edge_index, W1, att_src1, att_dst1, b1, gn_w, gn_b, gn_ms,
           W2, att_src2, att_dst2, b2):
    src = edge_index[0]
    dst = edge_index[1]

    xpad = jnp.zeros((_NP, _DIN), jnp.float32).at[:_N].set(x)
    am1 = _build_att_mat(att_src1, att_dst1, _H, _DIN)
    xw1, dots1 = _mm_att(xpad, W1, am1, b1.reshape(1, -1), _DIN)
    a_s1 = dots1[:, :_H].reshape(-1)
    a_d1 = dots1[:, _H:2 * _H].reshape(-1)

    out1 = _edge_sc1(src, dst, a_s1, a_d1, xw1)[0]

    st = _stats(out1)
    mean = st[0] / _N
    var = st[1] / _N - mean * mean * (2.0 * gn_ms - gn_ms * gn_ms)
    inv = gn_w / jnp.sqrt(var + 1e-5)
    scale = inv
    shift = gn_b - gn_ms * mean * inv

    am2 = _build_att_mat(att_src2, att_dst2, 1, _DOUT)
    xw2, dots2 = _norm_mm(out1, scale.reshape(1, -1), shift.reshape(1, -1),
                          W2, am2, b2.reshape(1, -1))
    a_s2 = dots2[:, 0]
    a_d2 = dots2[:, 1]

    out2 = _edge_sc2(src, dst, a_s2, a_d2, xw2)[0]
    return out2[:_N]


# speedup vs baseline: 4.0722x; 1.0001x over previous
import functools
import jax
import jax.numpy as jnp
from jax import lax
from jax.experimental import pallas as pl
from jax.experimental.pallas import tpu as pltpu
from jax.experimental.pallas import tpu_sc as plsc

_N = 10000
_E = 160000
_DIN = 256
_H = 4
_DOUT = 256
_CHUNK = 256
_CHUNK2 = 1024
_NP = 10240


def _leaky(v):
    return jnp.where(v > 0, v, 0.2 * v)


def _mm_att_body(x_ref, w_ref, am_ref, b_ref, xw_ref, dots_ref):
    xw_nob = jnp.dot(x_ref[...], w_ref[...], preferred_element_type=jnp.float32)
    dots_ref[...] = jnp.dot(xw_nob, am_ref[...],
                            preferred_element_type=jnp.float32)
    xw_ref[...] = xw_nob + b_ref[...]


def _mm_att(x, w, att_mat, brow, block_k):
    npad, k = x.shape
    kout = w.shape[1]
    grid = (npad // 128,)
    return pl.pallas_call(
        _mm_att_body,
        grid=grid,
        in_specs=[
            pl.BlockSpec((128, k), lambda i: (i, 0)),
            pl.BlockSpec((k, kout), lambda i: (0, 0)),
            pl.BlockSpec((kout, 128), lambda i: (0, 0)),
            pl.BlockSpec((1, kout), lambda i: (0, 0)),
        ],
        out_specs=[
            pl.BlockSpec((128, kout), lambda i: (i, 0)),
            pl.BlockSpec((128, 128), lambda i: (i, 0)),
        ],
        out_shape=[
            jax.ShapeDtypeStruct((npad, kout), jnp.float32),
            jax.ShapeDtypeStruct((npad, 128), jnp.float32),
        ],
    )(x, w, att_mat, brow)


def _stats_body(x_ref, o_ref):
    i = pl.program_id(0)
    rows = lax.broadcasted_iota(jnp.int32, (128, 1), 0) + i * 128
    t = jnp.where(rows < _N, x_ref[...], 0.0)
    s0 = jnp.sum(t, axis=0, keepdims=True)
    s1 = jnp.sum(t * t, axis=0, keepdims=True)

    @pl.when(i == 0)
    def _():
        o_ref[...] = jnp.zeros_like(o_ref)

    o_ref[0:1, :] += s0
    o_ref[1:2, :] += s1


def _stats(x):
    npad, d = x.shape
    return pl.pallas_call(
        _stats_body,
        grid=(npad // 128,),
        in_specs=[pl.BlockSpec((128, d), lambda i: (i, 0))],
        out_specs=pl.BlockSpec((8, d), lambda i: (0, 0)),
        out_shape=jax.ShapeDtypeStruct((8, d), jnp.float32),
    )(x)


def _norm_mm_body(x_ref, sc_ref, sh_ref, w_ref, am_ref, b_ref, xw_ref,
                  dots_ref):
    h = jnp.maximum(x_ref[...] * sc_ref[...] + sh_ref[...], 0.0)
    xw_nob = jnp.dot(h, w_ref[...], preferred_element_type=jnp.float32)
    dots_ref[...] = jnp.dot(xw_nob, am_ref[...],
                            preferred_element_type=jnp.float32)
    xw_ref[...] = xw_nob + b_ref[...]


def _norm_mm(x, scale, shift, w, att_mat, brow):
    npad, k = x.shape
    kout = w.shape[1]
    return pl.pallas_call(
        _norm_mm_body,
        grid=(npad // 128,),
        in_specs=[
            pl.BlockSpec((128, k), lambda i: (i, 0)),
            pl.BlockSpec((1, k), lambda i: (0, 0)),
            pl.BlockSpec((1, k), lambda i: (0, 0)),
            pl.BlockSpec((k, kout), lambda i: (0, 0)),
            pl.BlockSpec((kout, 128), lambda i: (0, 0)),
            pl.BlockSpec((1, kout), lambda i: (0, 0)),
        ],
        out_specs=[
            pl.BlockSpec((128, kout), lambda i: (i, 0)),
            pl.BlockSpec((128, 128), lambda i: (i, 0)),
        ],
        out_shape=[
            jax.ShapeDtypeStruct((npad, kout), jnp.float32),
            jax.ShapeDtypeStruct((npad, 128), jnp.float32),
        ],
    )(x, scale, shift, w, att_mat, brow)


def _leaky_exp(v):
    return jnp.exp(jnp.where(v > 0, v, 0.2 * v))


def _make_edge_sc(heads, dout, chunk):
    hd = heads * dout
    ept = _E // 16
    ng = ept // 16
    own = chunk // 16
    sh = chunk * heads
    n_pass = _NP // (2 * chunk)
    pub = ept + 16
    blk = 1024
    mesh = plsc.VectorSubcoreMesh(core_axis_name="c", subcore_axis_name="s")

    @functools.partial(
        pl.kernel, mesh=mesh,
        out_type=[jax.ShapeDtypeStruct((_NP, hd), jnp.float32),
                  jax.ShapeDtypeStruct((32 * sh,), jnp.float32),
                  jax.ShapeDtypeStruct((32 * pub,), jnp.int32),
                  jax.ShapeDtypeStruct((32 * pub,), jnp.int32),
                  jax.ShapeDtypeStruct((32 * 16,), jnp.int32)],
        compiler_params=pltpu.CompilerParams(needs_layout_passes=False),
        scratch_types=[
            pltpu.VMEM((ept,), jnp.int32),
            pltpu.VMEM((ept,), jnp.int32),
            pltpu.VMEM((_NP * heads,), jnp.float32),
            pltpu.VMEM((sh,), jnp.float32),
            pltpu.VMEM((sh,), jnp.float32),
            pltpu.VMEM((pub,), jnp.int32),
            pltpu.VMEM((pub,), jnp.int32),
            pltpu.VMEM((16, hd), jnp.float32),
            pltpu.VMEM((own, hd), jnp.float32),
            pltpu.VMEM((own * heads + 16,), jnp.float32),
            pltpu.VMEM((16 * heads + 16,), jnp.float32),
            pltpu.VMEM((2048,), jnp.float32),
            pltpu.VMEM((128,), jnp.float32),
            pltpu.VMEM((16,), jnp.int32),
            pltpu.VMEM((256,), jnp.int32),
            pltpu.VMEM((blk,), jnp.int32),
            pltpu.VMEM((blk,), jnp.int32),
            pltpu.VMEM((blk + 16,), jnp.int32),
            pltpu.VMEM((blk + 32,), jnp.int32),
            pltpu.VMEM_SHARED((sh,), jnp.float32),
            pltpu.SemaphoreType.DMA,
        ])
    def k(src_h, dst_h, as_h, ad_h, xw_h,
          out_h, parts_h, pubs_h, pubd_h, cnts_h,
          src_t, dst_t, as_t, ad_t, rs_t, srcc, dstc, rows, out_own,
          aws, awb, mbuf, vbuf, cbuf, cntbuf, psrc, pdst, osrc, orel,
          s_sh, sem):
        cid = lax.axis_index("c")
        sid = lax.axis_index("s")
        ebase = sid * ept
        pltpu.sync_copy(src_h.at[pl.ds(ebase, ept)], src_t)
        pltpu.sync_copy(dst_h.at[pl.ds(ebase, ept)], dst_t)
        pltpu.sync_copy(as_h, as_t)

        zi = jnp.zeros((16,), jnp.int32)
        zf = jnp.zeros((16,), jnp.float32)

        def zb(i, c):
            srcc[pl.ds(i * 16, 16)] = zi
            dstc[pl.ds(i * 16, 16)] = zi
            return c
        lax.fori_loop(0, pub // 16, zb, 0)

        def zo(i, c):
            osrc[pl.ds(i * 16, 16)] = zi
            orel[pl.ds(i * 16, 16)] = zi
            return c
        lax.fori_loop(0, (blk + 16) // 16, zo, 0)

        def pass_body(p, carry):
            lo = (2 * p + cid) * chunk
            pltpu.sync_copy(ad_h.at[pl.ds(lo * heads, sh)], ad_t)

            def zs(i, c):
                rs_t[pl.ds(i * 16, 16)] = zf
                return c
            lax.fori_loop(0, sh // 16, zs, 0)

            def ea(i, cnt):
                sl = pl.ds(i * 16, 16)
                sv = src_t[sl]
                rel = dst_t[sl] - lo
                m = (rel >= 0) & (rel < chunk)
                pc = plsc.all_reduce_population_count(m)

                @pl.when(pc[0] > 0)
                def _():
                    relc = jnp.where(m, rel, 0)
                    svc = jnp.where(m, sv, 0)
                    for h in range(heads):
                        asv = plsc.load_gather(as_t, [svc * heads + h])
                        adv = plsc.load_gather(ad_t, [relc * heads + h])
                        e = _leaky_exp(asv + adv)
                        plsc.addupdate_scatter(rs_t, [relc * heads + h], e,
                                               mask=m)
                    plsc.store_compressed(srcc.at[pl.ds(cnt, 16)], svc,
                                          mask=m)
                    plsc.store_compressed(dstc.at[pl.ds(cnt, 16)], relc,
                                          mask=m)
                return cnt + pc[0]
            cnt = lax.fori_loop(0, ng, ea, jnp.int32(0))

            def slp(i, c):
                off = sid * (own * heads) + i * 16
                e = _leaky_exp(as_t[pl.ds(lo * heads + off, 16)]
                               + ad_t[pl.ds(off, 16)])
                rs_t[pl.ds(off, 16)] = rs_t[pl.ds(off, 16)] + e
                return c
            lax.fori_loop(0, (own * heads) // 16, slp, 0)

            n_sl = sh // 128
            wid = cid * 16 + sid
            pltpu.sync_copy(rs_t, parts_h.at[pl.ds(wid * sh, sh)])
            plsc.subcore_barrier()
            for q in range((n_sl + 15) // 16):
                s_idx = sid + q * 16

                @pl.when(s_idx < n_sl)
                def _():
                    cps = [
                        pltpu.async_copy(
                            parts_h.at[pl.ds((cid * 16 + t) * sh
                                             + s_idx * 128, 128)],
                            mbuf.at[pl.ds(t * 128, 128)], sem)
                        for t in range(16)
                    ]
                    for cp in cps:
                        cp.wait()

                    def red(i, c):
                        sl = pl.ds(i * 16, 16)
                        acc = mbuf[pl.ds(i * 16, 16)]
                        for t in range(1, 16):
                            acc = acc + mbuf[pl.ds(t * 128 + i * 16, 16)]
                        vbuf[sl] = 1.0 / (acc + 1e-16)
                        return c
                    lax.fori_loop(0, 8, red, 0)
                    pltpu.sync_copy(vbuf, s_sh.at[pl.ds(s_idx * 128, 128)])
            plsc.subcore_barrier()
            pltpu.sync_copy(s_sh, rs_t)

            def swb(i, c):
                off = sid * (own * heads) + i * 16
                e = _leaky_exp(as_t[pl.ds(lo * heads + off, 16)]
                               + ad_t[pl.ds(off, 16)])
                aws[pl.ds(i * 16, 16)] = e * rs_t[pl.ds(off, 16)]
                return c
            lax.fori_loop(0, (own * heads) // 16, swb, 0)

            wid2 = cid * 16 + sid
            pbase = wid2 * pub

            def pb(b, c):
                pltpu.sync_copy(srcc.at[pl.ds(b * blk, blk)],
                                pubs_h.at[pl.ds(pbase + b * blk, blk)])
                pltpu.sync_copy(dstc.at[pl.ds(b * blk, blk)],
                                pubd_h.at[pl.ds(pbase + b * blk, blk)])
                return c
            lax.fori_loop(0, (cnt + blk - 1) // blk, pb, 0)
            cbuf[...] = jnp.broadcast_to(cnt, (16,))
            pltpu.sync_copy(cbuf, cnts_h.at[pl.ds(wid2 * 16, 16)])
            plsc.subcore_barrier()

            r0 = lo + sid * own

            def ib(g, c):
                pltpu.sync_copy(xw_h.at[pl.ds(r0 + g * 16, 16)], rows)

                def scale_row(rr, c2):
                    for h in range(heads):
                        wl = aws[pl.ds((g * 16 + rr) * heads + h, 16)]
                        wv = wl[0]
                        for cc in range(dout // 16):
                            sl2 = pl.ds(h * dout + cc * 16, 16)
                            out_own[g * 16 + rr, sl2] = rows[rr, sl2] * wv
                    return c2
                lax.fori_loop(0, 16, scale_row, 0)
                return c
            lax.fori_loop(0, own // 16, ib, 0)

            olo = sid * own
            pltpu.sync_copy(cnts_h.at[pl.ds(cid * 256, 256)], cntbuf)

            def per_src_tile(t, c):
                ctv = cntbuf[pl.ds(t * 16, 16)]
                cntt = ctv[0]
                base = (cid * 16 + t) * pub

                def per_blk(b, c2):
                    cp1 = pltpu.async_copy(
                        pubs_h.at[pl.ds(base + b * blk, blk)], psrc, sem)
                    cp2 = pltpu.async_copy(
                        pubd_h.at[pl.ds(base + b * blk, blk)], pdst, sem)
                    cp1.wait()
                    cp2.wait()
                    eib = jnp.minimum(cntt - b * blk, blk)

                    def fl(g, oc):
                        sl = pl.ds(g * 16, 16)
                        relv = pdst[sl]
                        srcv = psrc[sl]
                        lane = lax.iota(jnp.int32, 16)
                        m3 = ((g * 16 + lane) < eib) & (relv >= olo) \
                            & (relv < olo + own)
                        pc3 = plsc.all_reduce_population_count(m3)

                        @pl.when(pc3[0] > 0)
                        def _():
                            plsc.store_compressed(
                                osrc.at[pl.ds(oc, 16)],
                                jnp.where(m3, srcv, 0), mask=m3)
                            plsc.store_compressed(
                                orel.at[pl.ds(oc, 16)],
                                jnp.where(m3, relv - olo, 0), mask=m3)
                        return oc + pc3[0]
                    ocnt = lax.fori_loop(0, (eib + 15) // 16, fl,
                                         jnp.int32(0))

                    def eb(g, c3):
                        off = g * 16
                        sl = pl.ds(off, 16)
                        sv = osrc[sl]
                        rel = orel[sl]
                        gcp = pltpu.async_copy(xw_h.at[sv], rows, sem)
                        lane = lax.iota(jnp.int32, 16)
                        m2 = (off + lane) < ocnt
                        for h in range(heads):
                            asv = plsc.load_gather(as_t, [sv * heads + h])
                            adv = plsc.load_gather(
                                ad_t, [(rel + olo) * heads + h])
                            rsv = plsc.load_gather(
                                rs_t, [(rel + olo) * heads + h])
                            e = _leaky_exp(asv + adv)
                            awb[pl.ds(h * 16, 16)] = jnp.where(
                                m2, e * rsv, 0.0)
                        gcp.wait()

                        def acc_row(rr, c4):
                            rv = orel[pl.ds(off + rr, 16)]
                            rloc = rv[0]
                            for h in range(heads):
                                wl = awb[pl.ds(h * 16 + rr, 16)]
                                wv = wl[0]
                                for cc in range(dout // 16):
                                    sl2 = pl.ds(h * dout + cc * 16, 16)
                                    out_own[rloc, sl2] = (
                                        out_own[rloc, sl2]
                                        + rows[rr, sl2] * wv)
                            return c4
                        lax.fori_loop(0, 16, acc_row, 0)
                        return c3
                    lax.fori_loop(0, (ocnt + 15) // 16, eb, 0)
                    return c2
                lax.fori_loop(0, (cntt + blk - 1) // blk, per_blk, 0)
                return c
            lax.fori_loop(0, 16, per_src_tile, 0)

            pltpu.sync_copy(out_own, out_h.at[pl.ds(r0, own)])
            plsc.subcore_barrier()
            return carry
        lax.fori_loop(0, n_pass, pass_body, 0)

    return k


_edge_sc1 = _make_edge_sc(_H, _DIN, _CHUNK)
_edge_sc2 = _make_edge_sc(1, _DOUT, _CHUNK2)


def _edge_phase_xla(xw, a_s, a_d, src, dst, heads, dout):
    n = _N
    loop = jnp.arange(n, dtype=src.dtype)
    src_a = jnp.concatenate([src, loop])
    dst_a = jnp.concatenate([dst, loop])
    alpha = _leaky(a_s[src_a] + a_d[dst_a])
    e = jnp.exp(alpha)
    s = jax.ops.segment_sum(e, dst_a, num_segments=n)
    a = e / (s[dst_a] + 1e-16)
    xw3 = xw[:n].reshape(n, heads, dout)
    out = jax.ops.segment_sum(xw3[src_a] * a[:, :, None], dst_a,
                              num_segments=n)
    return out.reshape(n, heads * dout)


def _build_att_mat(att_src, att_dst, heads, dout):
    k = heads * dout
    m = jnp.zeros((k, 128), jnp.float32)
    rows = jnp.arange(k)
    m = m.at[rows, rows // dout].set(att_src.reshape(-1))
    m = m.at[rows, heads + rows // dout].set(att_dst.reshape(-1))
    return m


def kernel(x, edge_index, W1, att_src1, att_dst1, b1, gn_w, gn_b, gn_ms,
           W2, att_src2, att_dst2, b2):
    src = edge_index[0]
    dst = edge_index[1]

    xpad = jnp.zeros((_NP, _DIN), jnp.float32).at[:_N].set(x)
    am1 = _build_att_mat(att_src1, att_dst1, _H, _DIN)
    xw1, dots1 = _mm_att(xpad, W1, am1, b1.reshape(1, -1), _DIN)
    a_s1 = dots1[:, :_H].reshape(-1)
    a_d1 = dots1[:, _H:2 * _H].reshape(-1)

    out1 = _edge_sc1(src, dst, a_s1, a_d1, xw1)[0]

    st = _stats(out1)
    mean = st[0] / _N
    var = st[1] / _N - mean * mean * (2.0 * gn_ms - gn_ms * gn_ms)
    inv = gn_w / jnp.sqrt(var + 1e-5)
    scale = inv
    shift = gn_b - gn_ms * mean * inv

    am2 = _build_att_mat(att_src2, att_dst2, 1, _DOUT)
    xw2, dots2 = _norm_mm(out1, scale.reshape(1, -1), shift.reshape(1, -1),
                          W2, am2, b2.reshape(1, -1))
    a_s2 = dots2[:, 0]
    a_d2 = dots2[:, 1]

    out2 = _edge_sc2(src, dst, a_s2, a_d2, xw2)[0]
    return out2[:_N]


# speedup vs baseline: 4.0738x; 1.0004x over previous
import functools
import jax
import jax.numpy as jnp
from jax import lax
from jax.experimental import pallas as pl
from jax.experimental.pallas import tpu as pltpu
from jax.experimental.pallas import tpu_sc as plsc

_N = 10000
_E = 160000
_DIN = 256
_H = 4
_DOUT = 256
_CHUNK = 256
_CHUNK2 = 1024
_NP = 10240


def _mm_att_body(x_ref, w_ref, am_ref, b_ref, xw_ref, dots_ref):
    xw_nob = jnp.dot(x_ref[...], w_ref[...], preferred_element_type=jnp.float32)
    dots_ref[...] = jnp.dot(xw_nob, am_ref[...],
                            preferred_element_type=jnp.float32)
    xw_ref[...] = xw_nob + b_ref[...]


def _mm_att(x, w, att_mat, brow, block_k):
    npad, k = x.shape
    kout = w.shape[1]
    grid = (npad // 128,)
    return pl.pallas_call(
        _mm_att_body,
        grid=grid,
        in_specs=[
            pl.BlockSpec((128, k), lambda i: (i, 0)),
            pl.BlockSpec((k, kout), lambda i: (0, 0)),
            pl.BlockSpec((kout, 128), lambda i: (0, 0)),
            pl.BlockSpec((1, kout), lambda i: (0, 0)),
        ],
        out_specs=[
            pl.BlockSpec((128, kout), lambda i: (i, 0)),
            pl.BlockSpec((128, 128), lambda i: (i, 0)),
        ],
        out_shape=[
            jax.ShapeDtypeStruct((npad, kout), jnp.float32),
            jax.ShapeDtypeStruct((npad, 128), jnp.float32),
        ],
    )(x, w, att_mat, brow)


def _stats_body(x_ref, o_ref):
    i = pl.program_id(0)
    rows = lax.broadcasted_iota(jnp.int32, (128, 1), 0) + i * 128
    t = jnp.where(rows < _N, x_ref[...], 0.0)
    s0 = jnp.sum(t, axis=0, keepdims=True)
    s1 = jnp.sum(t * t, axis=0, keepdims=True)

    @pl.when(i == 0)
    def _():
        o_ref[...] = jnp.zeros_like(o_ref)

    o_ref[0:1, :] += s0
    o_ref[1:2, :] += s1


def _stats(x):
    npad, d = x.shape
    return pl.pallas_call(
        _stats_body,
        grid=(npad // 128,),
        in_specs=[pl.BlockSpec((128, d), lambda i: (i, 0))],
        out_specs=pl.BlockSpec((8, d), lambda i: (0, 0)),
        out_shape=jax.ShapeDtypeStruct((8, d), jnp.float32),
    )(x)


def _norm_mm_body(x_ref, sc_ref, sh_ref, w_ref, am_ref, b_ref, xw_ref,
                  dots_ref):
    h = jnp.maximum(x_ref[...] * sc_ref[...] + sh_ref[...], 0.0)
    xw_nob = jnp.dot(h, w_ref[...], preferred_element_type=jnp.float32)
    dots_ref[...] = jnp.dot(xw_nob, am_ref[...],
                            preferred_element_type=jnp.float32)
    xw_ref[...] = xw_nob + b_ref[...]


def _norm_mm(x, scale, shift, w, att_mat, brow):
    npad, k = x.shape
    kout = w.shape[1]
    return pl.pallas_call(
        _norm_mm_body,
        grid=(npad // 128,),
        in_specs=[
            pl.BlockSpec((128, k), lambda i: (i, 0)),
            pl.BlockSpec((1, k), lambda i: (0, 0)),
            pl.BlockSpec((1, k), lambda i: (0, 0)),
            pl.BlockSpec((k, kout), lambda i: (0, 0)),
            pl.BlockSpec((kout, 128), lambda i: (0, 0)),
            pl.BlockSpec((1, kout), lambda i: (0, 0)),
        ],
        out_specs=[
            pl.BlockSpec((128, kout), lambda i: (i, 0)),
            pl.BlockSpec((128, 128), lambda i: (i, 0)),
        ],
        out_shape=[
            jax.ShapeDtypeStruct((npad, kout), jnp.float32),
            jax.ShapeDtypeStruct((npad, 128), jnp.float32),
        ],
    )(x, scale, shift, w, att_mat, brow)


def _leaky_exp(v):
    return jnp.exp(jnp.where(v > 0, v, 0.2 * v))


def _make_edge_sc(heads, dout, chunk):
    hd = heads * dout
    ept = _E // 16
    ng = ept // 16
    own = chunk // 16
    sh = chunk * heads
    n_pass = _NP // (2 * chunk)
    pub = ept + 16
    blk = 1024
    mesh = plsc.VectorSubcoreMesh(core_axis_name="c", subcore_axis_name="s")

    @functools.partial(
        pl.kernel, mesh=mesh,
        out_type=[jax.ShapeDtypeStruct((_NP, hd), jnp.float32),
                  jax.ShapeDtypeStruct((32 * sh,), jnp.float32),
                  jax.ShapeDtypeStruct((32 * pub,), jnp.int32),
                  jax.ShapeDtypeStruct((32 * pub,), jnp.int32),
                  jax.ShapeDtypeStruct((32 * 16,), jnp.int32)],
        compiler_params=pltpu.CompilerParams(needs_layout_passes=False),
        scratch_types=[
            pltpu.VMEM((ept,), jnp.int32),
            pltpu.VMEM((ept,), jnp.int32),
            pltpu.VMEM((_NP * heads,), jnp.float32),
            pltpu.VMEM((sh,), jnp.float32),
            pltpu.VMEM((sh,), jnp.float32),
            pltpu.VMEM((pub,), jnp.int32),
            pltpu.VMEM((pub,), jnp.int32),
            pltpu.VMEM((16, hd), jnp.float32),
            pltpu.VMEM((own, hd), jnp.float32),
            pltpu.VMEM((own * heads + 16,), jnp.float32),
            pltpu.VMEM((16 * heads + 16,), jnp.float32),
            pltpu.VMEM((2048,), jnp.float32),
            pltpu.VMEM((128,), jnp.float32),
            pltpu.VMEM((16,), jnp.int32),
            pltpu.VMEM((256,), jnp.int32),
            pltpu.VMEM((blk,), jnp.int32),
            pltpu.VMEM((blk,), jnp.int32),
            pltpu.VMEM((blk + 16,), jnp.int32),
            pltpu.VMEM((blk + 32,), jnp.int32),
            pltpu.VMEM_SHARED((sh,), jnp.float32),
            pltpu.SemaphoreType.DMA,
        ])
    def k(src_h, dst_h, as_h, ad_h, xw_h,
          out_h, parts_h, pubs_h, pubd_h, cnts_h,
          src_t, dst_t, as_t, ad_t, rs_t, srcc, dstc, rows, out_own,
          aws, awb, mbuf, vbuf, cbuf, cntbuf, psrc, pdst, osrc, orel,
          s_sh, sem):
        cid = lax.axis_index("c")
        sid = lax.axis_index("s")
        ebase = sid * ept
        pltpu.sync_copy(src_h.at[pl.ds(ebase, ept)], src_t)
        pltpu.sync_copy(dst_h.at[pl.ds(ebase, ept)], dst_t)
        pltpu.sync_copy(as_h, as_t)

        zi = jnp.zeros((16,), jnp.int32)
        zf = jnp.zeros((16,), jnp.float32)

        def zb(i, c):
            srcc[pl.ds(i * 16, 16)] = zi
            dstc[pl.ds(i * 16, 16)] = zi
            return c
        lax.fori_loop(0, pub // 16, zb, 0)

        def zo(i, c):
            osrc[pl.ds(i * 16, 16)] = zi
            orel[pl.ds(i * 16, 16)] = zi
            return c
        lax.fori_loop(0, (blk + 16) // 16, zo, 0)

        def pass_body(p, carry):
            lo = (2 * p + cid) * chunk
            pltpu.sync_copy(ad_h.at[pl.ds(lo * heads, sh)], ad_t)

            def zs(i, c):
                rs_t[pl.ds(i * 16, 16)] = zf
                return c
            lax.fori_loop(0, sh // 16, zs, 0)

            def ea(i, cnt):
                sl = pl.ds(i * 16, 16)
                sv = src_t[sl]
                rel = dst_t[sl] - lo
                m = (rel >= 0) & (rel < chunk)
                pc = plsc.all_reduce_population_count(m)

                @pl.when(pc[0] > 0)
                def _():
                    relc = jnp.where(m, rel, 0)
                    svc = jnp.where(m, sv, 0)
                    for h in range(heads):
                        asv = plsc.load_gather(as_t, [svc * heads + h])
                        adv = plsc.load_gather(ad_t, [relc * heads + h])
                        e = _leaky_exp(asv + adv)
                        plsc.addupdate_scatter(rs_t, [relc * heads + h], e,
                                               mask=m)
                    plsc.store_compressed(srcc.at[pl.ds(cnt, 16)], svc,
                                          mask=m)
                    plsc.store_compressed(dstc.at[pl.ds(cnt, 16)], relc,
                                          mask=m)
                return cnt + pc[0]
            cnt = lax.fori_loop(0, ng, ea, jnp.int32(0))

            def slp(i, c):
                off = sid * (own * heads) + i * 16
                e = _leaky_exp(as_t[pl.ds(lo * heads + off, 16)]
                               + ad_t[pl.ds(off, 16)])
                rs_t[pl.ds(off, 16)] = rs_t[pl.ds(off, 16)] + e
                return c
            lax.fori_loop(0, (own * heads) // 16, slp, 0)

            n_sl = sh // 128
            wid = cid * 16 + sid
            pltpu.sync_copy(rs_t, parts_h.at[pl.ds(wid * sh, sh)])
            plsc.subcore_barrier()
            for q in range((n_sl + 15) // 16):
                s_idx = sid + q * 16

                @pl.when(s_idx < n_sl)
                def _():
                    cps = [
                        pltpu.async_copy(
                            parts_h.at[pl.ds((cid * 16 + t) * sh
                                             + s_idx * 128, 128)],
                            mbuf.at[pl.ds(t * 128, 128)], sem)
                        for t in range(16)
                    ]
                    for cp in cps:
                        cp.wait()

                    def red(i, c):
                        sl = pl.ds(i * 16, 16)
                        acc = mbuf[pl.ds(i * 16, 16)]
                        for t in range(1, 16):
                            acc = acc + mbuf[pl.ds(t * 128 + i * 16, 16)]
                        vbuf[sl] = 1.0 / (acc + 1e-16)
                        return c
                    lax.fori_loop(0, 8, red, 0)
                    pltpu.sync_copy(vbuf, s_sh.at[pl.ds(s_idx * 128, 128)])
            plsc.subcore_barrier()
            pltpu.sync_copy(s_sh, rs_t)

            def swb(i, c):
                off = sid * (own * heads) + i * 16
                e = _leaky_exp(as_t[pl.ds(lo * heads + off, 16)]
                               + ad_t[pl.ds(off, 16)])
                aws[pl.ds(i * 16, 16)] = e * rs_t[pl.ds(off, 16)]
                return c
            lax.fori_loop(0, (own * heads) // 16, swb, 0)

            wid2 = cid * 16 + sid
            pbase = wid2 * pub

            def pb(b, c):
                pltpu.sync_copy(srcc.at[pl.ds(b * blk, blk)],
                                pubs_h.at[pl.ds(pbase + b * blk, blk)])
                pltpu.sync_copy(dstc.at[pl.ds(b * blk, blk)],
                                pubd_h.at[pl.ds(pbase + b * blk, blk)])
                return c
            lax.fori_loop(0, (cnt + blk - 1) // blk, pb, 0)
            cbuf[...] = jnp.broadcast_to(cnt, (16,))
            pltpu.sync_copy(cbuf, cnts_h.at[pl.ds(wid2 * 16, 16)])
            plsc.subcore_barrier()

            r0 = lo + sid * own

            def ib(g, c):
                pltpu.sync_copy(xw_h.at[pl.ds(r0 + g * 16, 16)], rows)

                def scale_row(rr, c2):
                    for h in range(heads):
                        wl = aws[pl.ds((g * 16 + rr) * heads + h, 16)]
                        wv = wl[0]
                        for cc in range(dout // 16):
                            sl2 = pl.ds(h * dout + cc * 16, 16)
                            out_own[g * 16 + rr, sl2] = rows[rr, sl2] * wv
                    return c2
                lax.fori_loop(0, 16, scale_row, 0)
                return c
            lax.fori_loop(0, own // 16, ib, 0)

            olo = sid * own
            pltpu.sync_copy(cnts_h.at[pl.ds(cid * 256, 256)], cntbuf)

            def per_src_tile(t, c):
                ctv = cntbuf[pl.ds(t * 16, 16)]
                cntt = ctv[0]
                base = (cid * 16 + t) * pub

                def per_blk(b, c2):
                    cp1 = pltpu.async_copy(
                        pubs_h.at[pl.ds(base + b * blk, blk)], psrc, sem)
                    cp2 = pltpu.async_copy(
                        pubd_h.at[pl.ds(base + b * blk, blk)], pdst, sem)
                    cp1.wait()
                    cp2.wait()
                    eib = jnp.minimum(cntt - b * blk, blk)

                    def fl(g, oc):
                        sl = pl.ds(g * 16, 16)
                        relv = pdst[sl]
                        srcv = psrc[sl]
                        lane = lax.iota(jnp.int32, 16)
                        m3 = ((g * 16 + lane) < eib) & (relv >= olo) \
                            & (relv < olo + own)
                        pc3 = plsc.all_reduce_population_count(m3)

                        @pl.when(pc3[0] > 0)
                        def _():
                            plsc.store_compressed(
                                osrc.at[pl.ds(oc, 16)],
                                jnp.where(m3, srcv, 0), mask=m3)
                            plsc.store_compressed(
                                orel.at[pl.ds(oc, 16)],
                                jnp.where(m3, relv - olo, 0), mask=m3)
                        return oc + pc3[0]
                    ocnt = lax.fori_loop(0, (eib + 15) // 16, fl,
                                         jnp.int32(0))

                    def eb(g, c3):
                        off = g * 16
                        sl = pl.ds(off, 16)
                        sv = osrc[sl]
                        rel = orel[sl]
                        gcp = pltpu.async_copy(xw_h.at[sv], rows, sem)
                        lane = lax.iota(jnp.int32, 16)
                        m2 = (off + lane) < ocnt
                        for h in range(heads):
                            asv = plsc.load_gather(as_t, [sv * heads + h])
                            adv = plsc.load_gather(
                                ad_t, [(rel + olo) * heads + h])
                            rsv = plsc.load_gather(
                                rs_t, [(rel + olo) * heads + h])
                            e = _leaky_exp(asv + adv)
                            awb[pl.ds(h * 16, 16)] = jnp.where(
                                m2, e * rsv, 0.0)
                        gcp.wait()

                        def acc_row(rr, c4):
                            rv = orel[pl.ds(off + rr, 16)]
                            rloc = rv[0]
                            for h in range(heads):
                                wl = awb[pl.ds(h * 16 + rr, 16)]
                                wv = wl[0]
                                for cc in range(dout // 16):
                                    sl2 = pl.ds(h * dout + cc * 16, 16)
                                    out_own[rloc, sl2] = (
                                        out_own[rloc, sl2]
                                        + rows[rr, sl2] * wv)
                            return c4
                        lax.fori_loop(0, 16, acc_row, 0)
                        return c3
                    lax.fori_loop(0, (ocnt + 15) // 16, eb, 0)
                    return c2
                lax.fori_loop(0, (cntt + blk - 1) // blk, per_blk, 0)
                return c
            lax.fori_loop(0, 16, per_src_tile, 0)

            pltpu.sync_copy(out_own, out_h.at[pl.ds(r0, own)])
            plsc.subcore_barrier()
            return carry
        lax.fori_loop(0, n_pass, pass_body, 0)

    return k


_edge_sc1 = _make_edge_sc(_H, _DIN, _CHUNK)
_edge_sc2 = _make_edge_sc(1, _DOUT, _CHUNK2)


def _build_att_mat(att_src, att_dst, heads, dout):
    k = heads * dout
    m = jnp.zeros((k, 128), jnp.float32)
    rows = jnp.arange(k)
    m = m.at[rows, rows // dout].set(att_src.reshape(-1))
    m = m.at[rows, heads + rows // dout].set(att_dst.reshape(-1))
    return m


def kernel(x, edge_index, W1, att_src1, att_dst1, b1, gn_w, gn_b, gn_ms,
           W2, att_src2, att_dst2, b2):
    src = edge_index[0]
    dst = edge_index[1]

    xpad = jnp.zeros((_NP, _DIN), jnp.float32).at[:_N].set(x)
    am1 = _build_att_mat(att_src1, att_dst1, _H, _DIN)
    xw1, dots1 = _mm_att(xpad, W1, am1, b1.reshape(1, -1), _DIN)
    a_s1 = dots1[:, :_H].reshape(-1)
    a_d1 = dots1[:, _H:2 * _H].reshape(-1)

    out1 = _edge_sc1(src, dst, a_s1, a_d1, xw1)[0]

    st = _stats(out1)
    mean = st[0] / _N
    var = st[1] / _N - mean * mean * (2.0 * gn_ms - gn_ms * gn_ms)
    inv = gn_w / jnp.sqrt(var + 1e-5)
    scale = inv
    shift = gn_b - gn_ms * mean * inv

    am2 = _build_att_mat(att_src2, att_dst2, 1, _DOUT)
    xw2, dots2 = _norm_mm(out1, scale.reshape(1, -1), shift.reshape(1, -1),
                          W2, am2, b2.reshape(1, -1))
    a_s2 = dots2[:, 0]
    a_d2 = dots2[:, 1]

    out2 = _edge_sc2(src, dst, a_s2, a_d2, xw2)[0]
    return out2[:_N]
